# Initial kernel scaffold; baseline (speedup 1.0000x reference)
import jax
import jax.numpy as jnp
from jax.experimental import pallas as pl


def kernel(vertices, input, W_l, b_l, d_l, gamma_l, beta_l, W_m0, b_m0, d_m0, gamma_m0, beta_m0, W_m1, b_m1, d_m1, gamma_m1, beta_m1, W_g0, b_g0, d_g0, gamma_g0, beta_g0, W_g1, b_g1, d_g1, gamma_g1, beta_g1, W_down, b_down):
    raise NotImplementedError("write your pallas kernel here")


# speedup vs baseline: 1.9729x; 1.9729x over previous
import jax
import jax.numpy as jnp
from jax.experimental import pallas as pl
from jax.experimental.pallas import tpu as pltpu

_DIM = 128
_BS, _V = 2, 1024
_ROWS = _BS * _V
_NEG = -3.0e38
_K = 100

_LVL_L = (8, 128, 5)
_LVL_M = (32, 32, 20)
_LVL_G = (128, 8, 100)


def _bn_relu(x, gamma, beta):
    mean = jnp.mean(x, axis=0, keepdims=True)
    var = jnp.mean((x - mean) * (x - mean), axis=0, keepdims=True)
    return jax.nn.relu(gamma * (x - mean) * jax.lax.rsqrt(var + 1e-5) + beta)


def _kern(vp_ref, vt_ref, x_ref,
          Wl, bl, dl, gl, el,
          Wm0, bm0, dm0, gm0, em0,
          Wm1, bm1, dm1, gm1, em1,
          Wg0, bg0, dg0, gg0, eg0,
          Wg1, bg1, dg1, gg1, eg1,
          Wd, bd,
          out_ref, nd_ref, idx_ref, act_ref):
    f32 = jnp.float32
    lidx = jax.lax.broadcasted_iota(jnp.int32, (_V, _V), 1)

    for b in range(_BS):
        vp = vp_ref[b]
        vt = vt_ref[b]
        inner = jnp.dot(vp.astype(jnp.bfloat16), vt.astype(jnp.bfloat16),
                        preferred_element_type=f32)
        quad_c = jnp.sum(vp * vp, axis=1, keepdims=True)
        quad_r = jnp.sum(vt * vt, axis=0, keepdims=True)
        nd_ref[...] = -((-2.0 * inner + quad_r) + quad_c)

        def _extract(nd):
            m = jnp.max(nd, axis=1, keepdims=True)
            sel = nd == m
            col = jnp.min(jnp.where(sel, lidx, _V * 2), axis=1, keepdims=True)
            return col

        nd0 = nd_ref[...]
        col0 = _extract(nd0)
        nd_ref[...] = jnp.where(lidx == col0, _NEG, nd0)

        slot_lane = jax.lax.broadcasted_iota(jnp.int32, (_V, _DIM), 1)

        def _topk_body(j, acc):
            nd = nd_ref[...]
            col = _extract(nd)
            nd_ref[...] = jnp.where(lidx == col, _NEG, nd)
            return acc + jnp.where(slot_lane == j, col, 0)

        acc0 = jnp.zeros((_V, _DIM), jnp.int32)
        idx_ref[b] = jax.lax.fori_loop(0, _K, _topk_body, acc0)

    def conv(fm, lvl, W, bia, d, gamma, beta):
        n_pad, T, n_real = lvl
        fo = jnp.dot(fm.astype(jnp.bfloat16), W[...].astype(jnp.bfloat16),
                     preferred_element_type=f32) + bia[...]
        dd = d[...]
        dnorm = jnp.sqrt(jnp.sum(dd * dd, axis=0, keepdims=True))
        sdn = dd / jnp.maximum(dnorm, 1e-12)

        for b in range(_BS):
            base = b * _V
            fo2b = jax.lax.slice(fo, (base, _DIM), (base + _V, 2 * _DIM))
            vpb = vp_ref[b]

            def tile_body(i, c):
                idxt = idx_ref[b, pl.ds(i * T, T), :n_pad]
                wio = jax.lax.broadcasted_iota(jnp.int32, (T, n_pad, _V), 2)
                oh = (idxt[:, :, None] == wio).astype(f32).reshape(T * n_pad, _V)
                fsg = jnp.dot(oh, fo2b, preferred_element_type=f32,
                              precision=jax.lax.Precision.HIGHEST)
                posg = jnp.dot(oh, vpb, preferred_element_type=f32,
                               precision=jax.lax.Precision.HIGHEST)
                vpos = vp_ref[b, pl.ds(i * T, T), :]
                direc = posg.reshape(T, n_pad, 8) - vpos[:, None, :]
                nrm = jnp.sqrt(jnp.sum(direc * direc, axis=2, keepdims=True))
                dn = direc / jnp.maximum(nrm, 1e-12)
                theta = jax.nn.relu(
                    jnp.dot(dn.reshape(T * n_pad, 8).astype(jnp.bfloat16),
                            sdn.astype(jnp.bfloat16),
                            preferred_element_type=f32))
                prod = (theta * fsg).reshape(T, n_pad, _DIM)
                slot = jax.lax.broadcasted_iota(jnp.int32, (T, n_pad, _DIM), 1)
                mx = jnp.max(jnp.where(slot < n_real, prod, _NEG), axis=1)
                act_ref[pl.ds(base + i * T, T), :] = mx
                return c

            jax.lax.fori_loop(0, _V // T, tile_body, 0)

        fc = jax.lax.slice(fo, (0, 0), (_ROWS, _DIM))
        return _bn_relu(fc + act_ref[...], gamma[...], beta[...])

    x0 = x_ref[...]
    f_l = conv(x0, _LVL_L, Wl, bl, dl, gl, el)
    f_m = conv(x0, _LVL_M, Wm0, bm0, dm0, gm0, em0)
    f_m = conv(f_m, _LVL_M, Wm1, bm1, dm1, gm1, em1)
    f_g = conv(x0, _LVL_G, Wg0, bg0, dg0, gg0, eg0)
    f_g = conv(f_g, _LVL_G, Wg1, bg1, dg1, gg1, eg1)
    f_g = conv(f_g, _LVL_G, Wg0, bg0, dg0, gg0, eg0)

    cat = jnp.concatenate((f_l, f_m, f_g), axis=1)
    out_ref[...] = jax.nn.relu(
        jnp.dot(cat.astype(jnp.bfloat16), Wd[...].astype(jnp.bfloat16),
                preferred_element_type=f32) + bd[...])


def kernel(vertices, input, W_l, b_l, d_l, gamma_l, beta_l,
           W_m0, b_m0, d_m0, gamma_m0, beta_m0,
           W_m1, b_m1, d_m1, gamma_m1, beta_m1,
           W_g0, b_g0, d_g0, gamma_g0, beta_g0,
           W_g1, b_g1, d_g1, gamma_g1, beta_g1,
           W_down, b_down):
    f32 = jnp.float32
    vp = jnp.pad(vertices, ((0, 0), (0, 0), (0, 5)))
    vt = jnp.transpose(vp, (0, 2, 1))
    x2d = input.reshape(_ROWS, _DIM)

    def prep(W, b, d, g, e):
        return (W, b.reshape(1, -1), jnp.pad(d, ((0, 5), (0, 0))),
                g.reshape(1, -1), e.reshape(1, -1))

    args = (vp, vt, x2d,
            *prep(W_l, b_l, d_l, gamma_l, beta_l),
            *prep(W_m0, b_m0, d_m0, gamma_m0, beta_m0),
            *prep(W_m1, b_m1, d_m1, gamma_m1, beta_m1),
            *prep(W_g0, b_g0, d_g0, gamma_g0, beta_g0),
            *prep(W_g1, b_g1, d_g1, gamma_g1, beta_g1),
            W_down, b_down.reshape(1, -1))

    out = pl.pallas_call(
        _kern,
        out_shape=jax.ShapeDtypeStruct((_ROWS, 2 * _DIM), f32),
        scratch_shapes=[
            pltpu.VMEM((_V, _V), f32),
            pltpu.VMEM((_BS, _V, _DIM), jnp.int32),
            pltpu.VMEM((_ROWS, _DIM), f32),
        ],
    )(*args)
    return out.reshape(_BS, _V, 2 * _DIM)


# speedup vs baseline: 2.8725x; 1.4559x over previous
import jax
import jax.numpy as jnp
from jax.experimental import pallas as pl
from jax.experimental.pallas import tpu as pltpu

_DIM = 128
_BS, _V = 2, 1024
_ROWS = _BS * _V
_NEG = -3.0e38
_K = 100

_LVL_L = (8, 128, 5)
_LVL_M = (32, 32, 20)
_LVL_G = (128, 8, 100)


def _bn_relu(x, gamma, beta):
    mean = jnp.mean(x, axis=0, keepdims=True)
    var = jnp.mean((x - mean) * (x - mean), axis=0, keepdims=True)
    return jax.nn.relu(gamma * (x - mean) * jax.lax.rsqrt(var + 1e-5) + beta)


def _kern(vp_ref, vt_ref, x_ref,
          Wl, bl, dl, gl, el,
          Wm0, bm0, dm0, gm0, em0,
          Wm1, bm1, dm1, gm1, em1,
          Wg0, bg0, dg0, gg0, eg0,
          Wg1, bg1, dg1, gg1, eg1,
          Wd, bd,
          out_ref, nd_ref, idx_ref, act_ref):
    f32 = jnp.float32
    lidx = jax.lax.broadcasted_iota(jnp.int32, (_V, _V), 1)

    for b in range(_BS):
        vp = vp_ref[b]
        vt = vt_ref[b]
        inner = jnp.dot(vp.astype(jnp.bfloat16), vt.astype(jnp.bfloat16),
                        preferred_element_type=f32)
        quad_c = jnp.sum(vp * vp, axis=1, keepdims=True)
        quad_r = jnp.sum(vt * vt, axis=0, keepdims=True)
        nd_ref[...] = -((-2.0 * inner + quad_r) + quad_c)

        def _extract(nd):
            m = jnp.max(nd, axis=1, keepdims=True)
            sel = nd == m
            col = jnp.min(jnp.where(sel, lidx, _V * 2), axis=1, keepdims=True)
            return col

        nd0 = nd_ref[...]
        col0 = _extract(nd0)
        nd_ref[...] = jnp.where(lidx == col0, _NEG, nd0)

        slot_lane = jax.lax.broadcasted_iota(jnp.int32, (_V, _DIM), 1)

        def _topk_body(j, acc):
            nd = nd_ref[...]
            col = _extract(nd)
            nd_ref[...] = jnp.where(lidx == col, _NEG, nd)
            return acc + jnp.where(slot_lane == j, col, 0)

        acc0 = jnp.zeros((_V, _DIM), jnp.int32)
        idx_ref[b] = jax.lax.fori_loop(0, _K, _topk_body, acc0)

    def conv(fm, lvl, W, bia, d, gamma, beta):
        n_pad, T, n_real = lvl
        fo = jnp.dot(fm.astype(jnp.bfloat16), W[...].astype(jnp.bfloat16),
                     preferred_element_type=f32) + bia[...]
        dd = d[...]
        dnorm = jnp.sqrt(jnp.sum(dd * dd, axis=0, keepdims=True))
        sdn = dd / jnp.maximum(dnorm, 1e-12)

        for b in range(_BS):
            base = b * _V
            fo2b = jax.lax.slice(fo, (base, _DIM), (base + _V, 2 * _DIM))
            fo2_hi = fo2b.astype(jnp.bfloat16)
            fo2_lo = (fo2b - fo2_hi.astype(f32)).astype(jnp.bfloat16)
            vpb = vp_ref[b]
            vp_hi = vpb.astype(jnp.bfloat16)
            vp_lo = (vpb - vp_hi.astype(f32)).astype(jnp.bfloat16)

            def tile_body(i, c):
                idxt = idx_ref[b, pl.ds(i * T, T), :n_pad]
                wio = jax.lax.broadcasted_iota(jnp.int32, (T, n_pad, _V), 2)
                oh = (idxt[:, :, None] == wio).astype(
                    jnp.bfloat16).reshape(T * n_pad, _V)
                fsg = (jnp.dot(oh, fo2_hi, preferred_element_type=f32)
                       + jnp.dot(oh, fo2_lo, preferred_element_type=f32))
                posg = jnp.dot(oh.astype(f32), vpb, preferred_element_type=f32,
                               precision=jax.lax.Precision.HIGHEST)
                vpos = vp_ref[b, pl.ds(i * T, T), :]
                direc = posg.reshape(T, n_pad, 8) - vpos[:, None, :]
                nrm = jnp.sqrt(jnp.sum(direc * direc, axis=2, keepdims=True))
                dn = direc / jnp.maximum(nrm, 1e-12)
                theta = jax.nn.relu(
                    jnp.dot(dn.reshape(T * n_pad, 8).astype(jnp.bfloat16),
                            sdn.astype(jnp.bfloat16),
                            preferred_element_type=f32))
                prod = (theta * fsg).reshape(T, n_pad, _DIM)
                slot = jax.lax.broadcasted_iota(jnp.int32, (T, n_pad, _DIM), 1)
                mx = jnp.max(jnp.where(slot < n_real, prod, _NEG), axis=1)
                act_ref[pl.ds(base + i * T, T), :] = mx
                return c

            jax.lax.fori_loop(0, _V // T, tile_body, 0)

        fc = jax.lax.slice(fo, (0, 0), (_ROWS, _DIM))
        return _bn_relu(fc + act_ref[...], gamma[...], beta[...])

    x0 = x_ref[...]
    f_l = conv(x0, _LVL_L, Wl, bl, dl, gl, el)
    f_m = conv(x0, _LVL_M, Wm0, bm0, dm0, gm0, em0)
    f_m = conv(f_m, _LVL_M, Wm1, bm1, dm1, gm1, em1)
    f_g = conv(x0, _LVL_G, Wg0, bg0, dg0, gg0, eg0)
    f_g = conv(f_g, _LVL_G, Wg1, bg1, dg1, gg1, eg1)
    f_g = conv(f_g, _LVL_G, Wg0, bg0, dg0, gg0, eg0)

    cat = jnp.concatenate((f_l, f_m, f_g), axis=1)
    out_ref[...] = jax.nn.relu(
        jnp.dot(cat.astype(jnp.bfloat16), Wd[...].astype(jnp.bfloat16),
                preferred_element_type=f32) + bd[...])


def kernel(vertices, input, W_l, b_l, d_l, gamma_l, beta_l,
           W_m0, b_m0, d_m0, gamma_m0, beta_m0,
           W_m1, b_m1, d_m1, gamma_m1, beta_m1,
           W_g0, b_g0, d_g0, gamma_g0, beta_g0,
           W_g1, b_g1, d_g1, gamma_g1, beta_g1,
           W_down, b_down):
    f32 = jnp.float32
    vp = jnp.pad(vertices, ((0, 0), (0, 0), (0, 5)))
    vt = jnp.transpose(vp, (0, 2, 1))
    x2d = input.reshape(_ROWS, _DIM)

    def prep(W, b, d, g, e):
        return (W, b.reshape(1, -1), jnp.pad(d, ((0, 5), (0, 0))),
                g.reshape(1, -1), e.reshape(1, -1))

    args = (vp, vt, x2d,
            *prep(W_l, b_l, d_l, gamma_l, beta_l),
            *prep(W_m0, b_m0, d_m0, gamma_m0, beta_m0),
            *prep(W_m1, b_m1, d_m1, gamma_m1, beta_m1),
            *prep(W_g0, b_g0, d_g0, gamma_g0, beta_g0),
            *prep(W_g1, b_g1, d_g1, gamma_g1, beta_g1),
            W_down, b_down.reshape(1, -1))

    out = pl.pallas_call(
        _kern,
        out_shape=jax.ShapeDtypeStruct((_ROWS, 2 * _DIM), f32),
        scratch_shapes=[
            pltpu.VMEM((_V, _V), f32),
            pltpu.VMEM((_BS, _V, _DIM), jnp.int32),
            pltpu.VMEM((_ROWS, _DIM), f32),
        ],
    )(*args)
    return out.reshape(_BS, _V, 2 * _DIM)


# speedup vs baseline: 2.8744x; 1.0007x over previous
import jax
import jax.numpy as jnp
from jax.experimental import pallas as pl
from jax.experimental.pallas import tpu as pltpu

_DIM = 128
_BS, _V = 2, 1024
_ROWS = _BS * _V
_NEG = -3.0e38
_K = 100

_LVL_L = (8, 128, 5)
_LVL_M = (32, 32, 20)
_LVL_G = (128, 8, 100)


def _bn_relu(x, gamma, beta):
    mean = jnp.mean(x, axis=0, keepdims=True)
    var = jnp.mean((x - mean) * (x - mean), axis=0, keepdims=True)
    return jax.nn.relu(gamma * (x - mean) * jax.lax.rsqrt(var + 1e-5) + beta)


def _kern(vp_ref, vt_ref, x_ref,
          Wl, bl, dl, gl, el,
          Wm0, bm0, dm0, gm0, em0,
          Wm1, bm1, dm1, gm1, em1,
          Wg0, bg0, dg0, gg0, eg0,
          Wg1, bg1, dg1, gg1, eg1,
          Wd, bd,
          out_ref, nd_ref, idx_ref, act_ref):
    f32 = jnp.float32
    lidx = jax.lax.broadcasted_iota(jnp.int32, (_ROWS, _V), 1)

    for b in range(_BS):
        vp = vp_ref[b]
        vt = vt_ref[b]
        inner = jnp.dot(vp.astype(jnp.bfloat16), vt.astype(jnp.bfloat16),
                        preferred_element_type=f32)
        quad_c = jnp.sum(vp * vp, axis=1, keepdims=True)
        quad_r = jnp.sum(vt * vt, axis=0, keepdims=True)
        nd_ref[b * _V:(b + 1) * _V, :] = -((-2.0 * inner + quad_r) + quad_c)

    def _extract(nd):
        m = jnp.max(nd, axis=1, keepdims=True)
        sel = nd == m
        col = jnp.min(jnp.where(sel, lidx, _V * 2), axis=1, keepdims=True)
        return col

    nd0 = nd_ref[...]
    col0 = _extract(nd0)
    nd_ref[...] = jnp.where(lidx == col0, _NEG, nd0)

    slot_lane = jax.lax.broadcasted_iota(jnp.int32, (_ROWS, _DIM), 1)

    def _topk_body(j, acc):
        nd = nd_ref[...]
        col = _extract(nd)
        nd_ref[...] = jnp.where(lidx == col, _NEG, nd)
        return acc + jnp.where(slot_lane == j, col, 0)

    acc0 = jnp.zeros((_ROWS, _DIM), jnp.int32)
    idx_ref[...] = jax.lax.fori_loop(0, _K, _topk_body, acc0)

    def conv(fm, lvl, W, bia, d, gamma, beta):
        n_pad, T, n_real = lvl
        fo = jnp.dot(fm.astype(jnp.bfloat16), W[...].astype(jnp.bfloat16),
                     preferred_element_type=f32) + bia[...]
        dd = d[...]
        dnorm = jnp.sqrt(jnp.sum(dd * dd, axis=0, keepdims=True))
        sdn = dd / jnp.maximum(dnorm, 1e-12)

        for b in range(_BS):
            base = b * _V
            fo2b = jax.lax.slice(fo, (base, _DIM), (base + _V, 2 * _DIM))
            fo2_hi = fo2b.astype(jnp.bfloat16)
            fo2_lo = (fo2b - fo2_hi.astype(f32)).astype(jnp.bfloat16)
            vpb = vp_ref[b]
            vp_hi = vpb.astype(jnp.bfloat16)
            vp_lo = (vpb - vp_hi.astype(f32)).astype(jnp.bfloat16)

            def tile_body(i, c):
                idxt = idx_ref[pl.ds(base + i * T, T), :n_pad]
                wio = jax.lax.broadcasted_iota(jnp.int32, (T, n_pad, _V), 2)
                oh = (idxt[:, :, None] == wio).astype(
                    jnp.bfloat16).reshape(T * n_pad, _V)
                fsg = (jnp.dot(oh, fo2_hi, preferred_element_type=f32)
                       + jnp.dot(oh, fo2_lo, preferred_element_type=f32))
                posg = jnp.dot(oh.astype(f32), vpb, preferred_element_type=f32,
                               precision=jax.lax.Precision.HIGHEST)
                vpos = vp_ref[b, pl.ds(i * T, T), :]
                direc = posg.reshape(T, n_pad, 8) - vpos[:, None, :]
                nrm = jnp.sqrt(jnp.sum(direc * direc, axis=2, keepdims=True))
                dn = direc / jnp.maximum(nrm, 1e-12)
                theta = jax.nn.relu(
                    jnp.dot(dn.reshape(T * n_pad, 8).astype(jnp.bfloat16),
                            sdn.astype(jnp.bfloat16),
                            preferred_element_type=f32))
                prod = (theta * fsg).reshape(T, n_pad, _DIM)
                slot = jax.lax.broadcasted_iota(jnp.int32, (T, n_pad, _DIM), 1)
                mx = jnp.max(jnp.where(slot < n_real, prod, _NEG), axis=1)
                act_ref[pl.ds(base + i * T, T), :] = mx
                return c

            jax.lax.fori_loop(0, _V // T, tile_body, 0)

        fc = jax.lax.slice(fo, (0, 0), (_ROWS, _DIM))
        return _bn_relu(fc + act_ref[...], gamma[...], beta[...])

    x0 = x_ref[...]
    f_l = conv(x0, _LVL_L, Wl, bl, dl, gl, el)
    f_m = conv(x0, _LVL_M, Wm0, bm0, dm0, gm0, em0)
    f_m = conv(f_m, _LVL_M, Wm1, bm1, dm1, gm1, em1)
    f_g = conv(x0, _LVL_G, Wg0, bg0, dg0, gg0, eg0)
    f_g = conv(f_g, _LVL_G, Wg1, bg1, dg1, gg1, eg1)
    f_g = conv(f_g, _LVL_G, Wg0, bg0, dg0, gg0, eg0)

    cat = jnp.concatenate((f_l, f_m, f_g), axis=1)
    out_ref[...] = jax.nn.relu(
        jnp.dot(cat.astype(jnp.bfloat16), Wd[...].astype(jnp.bfloat16),
                preferred_element_type=f32) + bd[...])


def kernel(vertices, input, W_l, b_l, d_l, gamma_l, beta_l,
           W_m0, b_m0, d_m0, gamma_m0, beta_m0,
           W_m1, b_m1, d_m1, gamma_m1, beta_m1,
           W_g0, b_g0, d_g0, gamma_g0, beta_g0,
           W_g1, b_g1, d_g1, gamma_g1, beta_g1,
           W_down, b_down):
    f32 = jnp.float32
    vp = jnp.pad(vertices, ((0, 0), (0, 0), (0, 5)))
    vt = jnp.transpose(vp, (0, 2, 1))
    x2d = input.reshape(_ROWS, _DIM)

    def prep(W, b, d, g, e):
        return (W, b.reshape(1, -1), jnp.pad(d, ((0, 5), (0, 0))),
                g.reshape(1, -1), e.reshape(1, -1))

    args = (vp, vt, x2d,
            *prep(W_l, b_l, d_l, gamma_l, beta_l),
            *prep(W_m0, b_m0, d_m0, gamma_m0, beta_m0),
            *prep(W_m1, b_m1, d_m1, gamma_m1, beta_m1),
            *prep(W_g0, b_g0, d_g0, gamma_g0, beta_g0),
            *prep(W_g1, b_g1, d_g1, gamma_g1, beta_g1),
            W_down, b_down.reshape(1, -1))

    out = pl.pallas_call(
        _kern,
        out_shape=jax.ShapeDtypeStruct((_ROWS, 2 * _DIM), f32),
        scratch_shapes=[
            pltpu.VMEM((_ROWS, _V), f32),
            pltpu.VMEM((_ROWS, _DIM), jnp.int32),
            pltpu.VMEM((_ROWS, _DIM), f32),
        ],
    )(*args)
    return out.reshape(_BS, _V, 2 * _DIM)


# speedup vs baseline: 4.1941x; 1.4591x over previous
import jax
import jax.numpy as jnp
from jax.experimental import pallas as pl
from jax.experimental.pallas import tpu as pltpu

_DIM = 128
_BS, _V = 2, 1024
_ROWS = _BS * _V
_NEG = -3.0e38
_K = 100

_LVL_L = (8, 128, 5)
_LVL_M = (32, 32, 20)
_LVL_G = (128, 8, 100)


def _bn_relu(x, gamma, beta):
    mean = jnp.mean(x, axis=0, keepdims=True)
    var = jnp.mean((x - mean) * (x - mean), axis=0, keepdims=True)
    return jax.nn.relu(gamma * (x - mean) * jax.lax.rsqrt(var + 1e-5) + beta)


def _kern(vp_ref, vt_ref, x_ref,
          Wl, bl, dl, gl, el,
          Wm0, bm0, dm0, gm0, em0,
          Wm1, bm1, dm1, gm1, em1,
          Wg0, bg0, dg0, gg0, eg0,
          Wg1, bg1, dg1, gg1, eg1,
          Wd, bd,
          out_ref, nd_ref, idx_ref, act_ref):
    f32 = jnp.float32
    lidx = jax.lax.broadcasted_iota(jnp.int32, (_ROWS, _V), 1)

    for b in range(_BS):
        vp = vp_ref[b]
        vt = vt_ref[b]
        inner = jnp.dot(vp.astype(jnp.bfloat16), vt.astype(jnp.bfloat16),
                        preferred_element_type=f32)
        quad_c = jnp.sum(vp * vp, axis=1, keepdims=True)
        quad_r = jnp.sum(vt * vt, axis=0, keepdims=True)
        nd_ref[b * _V:(b + 1) * _V, :] = -((-2.0 * inner + quad_r) + quad_c)

    def _extract(nd):
        m = jnp.max(nd, axis=1, keepdims=True)
        sel = nd == m
        col = jnp.min(jnp.where(sel, lidx, _V * 2), axis=1, keepdims=True)
        return col

    nd0 = nd_ref[...]
    col0 = _extract(nd0)
    nd_ref[...] = jnp.where(lidx == col0, _NEG, nd0)

    slot_lane = jax.lax.broadcasted_iota(jnp.int32, (_ROWS, _DIM), 1)

    def _topk_body(j, acc):
        nd = nd_ref[...]
        col = _extract(nd)
        nd_ref[...] = jnp.where(lidx == col, _NEG, nd)
        return acc + jnp.where(slot_lane == j, col, 0)

    acc0 = jnp.zeros((_ROWS, _DIM), jnp.int32)
    idx_ref[...] = jax.lax.fori_loop(0, _K, _topk_body, acc0)

    def conv(fm, lvl, W, bia, d, gamma, beta):
        n_pad, T, n_real = lvl
        fo = jnp.dot(fm.astype(jnp.bfloat16), W[...].astype(jnp.bfloat16),
                     preferred_element_type=f32) + bia[...]
        dd = d[...]
        dnorm = jnp.sqrt(jnp.sum(dd * dd, axis=0, keepdims=True))
        sdn = dd / jnp.maximum(dnorm, 1e-12)

        for b in range(_BS):
            base = b * _V
            fo2b = jax.lax.slice(fo, (base, _DIM), (base + _V, 2 * _DIM))
            fo2_hi = fo2b.astype(jnp.bfloat16)
            fo2_lo = (fo2b - fo2_hi.astype(f32)).astype(jnp.bfloat16)
            vpb = vp_ref[b]
            vp_h1 = vpb.astype(jnp.bfloat16)
            r1 = vpb - vp_h1.astype(f32)
            vp_h2 = r1.astype(jnp.bfloat16)
            vp_h3 = (r1 - vp_h2.astype(f32)).astype(jnp.bfloat16)

            def tile_body(i, c):
                idxt = idx_ref[pl.ds(base + i * T, T), :n_pad]
                wio = jax.lax.broadcasted_iota(jnp.int32, (T, n_pad, _V), 2)
                oh = (idxt[:, :, None] == wio).astype(
                    jnp.bfloat16).reshape(T * n_pad, _V)
                fsg = (jnp.dot(oh, fo2_hi, preferred_element_type=f32)
                       + jnp.dot(oh, fo2_lo, preferred_element_type=f32))
                posg = (jnp.dot(oh, vp_h1, preferred_element_type=f32)
                        + jnp.dot(oh, vp_h2, preferred_element_type=f32)
                        + jnp.dot(oh, vp_h3, preferred_element_type=f32))
                vpos = vp_ref[b, pl.ds(i * T, T), :]
                direc = posg.reshape(T, n_pad, 8) - vpos[:, None, :]
                nrm = jnp.sqrt(jnp.sum(direc * direc, axis=2, keepdims=True))
                dn = direc / jnp.maximum(nrm, 1e-12)
                theta = jax.nn.relu(
                    jnp.dot(dn.reshape(T * n_pad, 8).astype(jnp.bfloat16),
                            sdn.astype(jnp.bfloat16),
                            preferred_element_type=f32))
                prod = (theta * fsg).reshape(T, n_pad, _DIM)
                slot = jax.lax.broadcasted_iota(jnp.int32, (T, n_pad, _DIM), 1)
                mx = jnp.max(jnp.where(slot < n_real, prod, _NEG), axis=1)
                act_ref[pl.ds(base + i * T, T), :] = mx
                return c

            jax.lax.fori_loop(0, _V // T, tile_body, 0)

        fc = jax.lax.slice(fo, (0, 0), (_ROWS, _DIM))
        return _bn_relu(fc + act_ref[...], gamma[...], beta[...])

    x0 = x_ref[...]
    f_l = conv(x0, _LVL_L, Wl, bl, dl, gl, el)
    f_m = conv(x0, _LVL_M, Wm0, bm0, dm0, gm0, em0)
    f_m = conv(f_m, _LVL_M, Wm1, bm1, dm1, gm1, em1)
    f_g = conv(x0, _LVL_G, Wg0, bg0, dg0, gg0, eg0)
    f_g = conv(f_g, _LVL_G, Wg1, bg1, dg1, gg1, eg1)
    f_g = conv(f_g, _LVL_G, Wg0, bg0, dg0, gg0, eg0)

    cat = jnp.concatenate((f_l, f_m, f_g), axis=1)
    out_ref[...] = jax.nn.relu(
        jnp.dot(cat.astype(jnp.bfloat16), Wd[...].astype(jnp.bfloat16),
                preferred_element_type=f32) + bd[...])


def kernel(vertices, input, W_l, b_l, d_l, gamma_l, beta_l,
           W_m0, b_m0, d_m0, gamma_m0, beta_m0,
           W_m1, b_m1, d_m1, gamma_m1, beta_m1,
           W_g0, b_g0, d_g0, gamma_g0, beta_g0,
           W_g1, b_g1, d_g1, gamma_g1, beta_g1,
           W_down, b_down):
    f32 = jnp.float32
    vp = jnp.pad(vertices, ((0, 0), (0, 0), (0, 5)))
    vt = jnp.transpose(vp, (0, 2, 1))
    x2d = input.reshape(_ROWS, _DIM)

    def prep(W, b, d, g, e):
        return (W, b.reshape(1, -1), jnp.pad(d, ((0, 5), (0, 0))),
                g.reshape(1, -1), e.reshape(1, -1))

    args = (vp, vt, x2d,
            *prep(W_l, b_l, d_l, gamma_l, beta_l),
            *prep(W_m0, b_m0, d_m0, gamma_m0, beta_m0),
            *prep(W_m1, b_m1, d_m1, gamma_m1, beta_m1),
            *prep(W_g0, b_g0, d_g0, gamma_g0, beta_g0),
            *prep(W_g1, b_g1, d_g1, gamma_g1, beta_g1),
            W_down, b_down.reshape(1, -1))

    out = pl.pallas_call(
        _kern,
        out_shape=jax.ShapeDtypeStruct((_ROWS, 2 * _DIM), f32),
        scratch_shapes=[
            pltpu.VMEM((_ROWS, _V), f32),
            pltpu.VMEM((_ROWS, _DIM), jnp.int32),
            pltpu.VMEM((_ROWS, _DIM), f32),
        ],
    )(*args)
    return out.reshape(_BS, _V, 2 * _DIM)
